# baseline (speedup 1.0000x reference)
import functools

import jax
import jax.numpy as jnp
from jax.experimental import pallas as pl
from jax.experimental.pallas import tpu as pltpu

_B, _S, _H = 4, 2048, 4096
_E = 64
_TOPK = 8
_AUX_COEF = 0.01
_Z_COEF = 0.001


def _router_block_kernel(x_ref, w_ref, logits_ref, mask_ref, comb_ref,
                         idx_ref, accp_ref, acct_ref, scal_ref, acc_ref,
                         *, nblocks, nh, topk, aux_coef, z_coef, n_tokens):
    i = pl.program_id(0)
    j = pl.program_id(1)
    e = w_ref.shape[0]
    hc = x_ref.shape[1]

    part = jax.lax.dot_general(
        w_ref[:, pl.ds(j * hc, hc)], x_ref[...],
        (((1,), (1,)), ((), ())), preferred_element_type=jnp.float32)

    @pl.when(j == 0)
    def _seed():
        acc_ref[...] = part

    @pl.when(j != 0)
    def _acc():
        acc_ref[...] += part

    @pl.when(j == nh - 1)
    def _tail():
        lt = acc_ref[...]
        logits_ref[...] = lt.T

        tb = lt.shape[1]
        iota_e = jax.lax.broadcasted_iota(jnp.int32, (e, tb), 0)

        m = jnp.max(lt, axis=0, keepdims=True)
        ex = jnp.exp(lt - m)
        se = jnp.sum(ex, axis=0, keepdims=True)
        probs = ex / se
        lse = m + jnp.log(se)
        z_c = jnp.sum(lse * lse)
        ent_c = jnp.sum(lse) - jnp.sum(probs * lt)

        work = lt
        vals = []
        sels = []
        mask_t = jnp.zeros((e, tb), jnp.float32)
        idx_rows = []
        for _ in range(topk):
            mk = jnp.max(work, axis=0, keepdims=True)
            is_max = work == mk
            idx_k = jnp.min(jnp.where(is_max, iota_e, e), axis=0,
                            keepdims=True)
            sel = iota_e == idx_k
            work = jnp.where(sel, -jnp.inf, work)
            vals.append(mk)
            sels.append(sel)
            idx_rows.append(idx_k)
            mask_t = mask_t + sel.astype(jnp.float32)

        idx_ref[...] = jnp.concatenate(idx_rows, axis=0)
        mask_ref[...] = mask_t.T

        topv = jnp.concatenate(vals, axis=0)
        tex = jnp.exp(topv - topv[:1])
        tse = jnp.sum(tex, axis=0, keepdims=True)
        rw = tex / tse
        comb_t = jnp.zeros((e, tb), jnp.float32)
        for k in range(topk):
            comb_t = comb_t + jnp.where(sels[k], rw[k:k + 1, :], 0.0)
        comb_ref[...] = comb_t.T

        @pl.when(i == 0)
        def _init():
            accp_ref[...] = jnp.zeros_like(accp_ref)
            acct_ref[...] = jnp.zeros_like(acct_ref)
            scal_ref[...] = jnp.zeros_like(scal_ref)

        accp_ref[...] += jnp.sum(probs, axis=1, keepdims=True)
        acct_ref[...] += jnp.sum(mask_t, axis=1, keepdims=True)
        lane = jax.lax.broadcasted_iota(jnp.int32, scal_ref.shape, 1)
        scal_ref[...] += (jnp.where(lane == 0, z_c, 0.0)
                          + jnp.where(lane == 1, ent_c, 0.0))

        @pl.when(i == nblocks - 1)
        def _finalize():
            avg_probs = accp_ref[...] / n_tokens
            tpe = acct_ref[...]
            total = jnp.sum(tpe)
            aux = jnp.sum(avg_probs * (tpe / (total + 1e-6))) * e * aux_coef
            srow = scal_ref[...]
            sum_z = jnp.sum(jnp.where(lane == 0, srow, 0.0))
            sum_ent = jnp.sum(jnp.where(lane == 1, srow, 0.0))
            scal_ref[...] = (
                jnp.where(lane == 2, aux, 0.0)
                + jnp.where(lane == 3, sum_z / n_tokens * z_coef, 0.0)
                + jnp.where(lane == 4, sum_ent / n_tokens, 0.0))


@functools.partial(jax.jit, static_argnames=("block_tokens", "block_h"))
def _router(hidden_states, w_gate, block_tokens=1024, block_h=1024):
    b, s, h = hidden_states.shape
    e = w_gate.shape[0]
    t = b * s
    tb = block_tokens
    hc = block_h
    nblocks = t // tb
    nh = h // hc
    x = hidden_states.reshape(t, h)

    body = functools.partial(
        _router_block_kernel, nblocks=nblocks, nh=nh, topk=_TOPK,
        aux_coef=_AUX_COEF, z_coef=_Z_COEF, n_tokens=float(t))

    out_shape = [
        jax.ShapeDtypeStruct((t, e), jnp.float32),
        jax.ShapeDtypeStruct((t, e), jnp.float32),
        jax.ShapeDtypeStruct((t, e), jnp.float32),
        jax.ShapeDtypeStruct((_TOPK, t), jnp.int32),
        jax.ShapeDtypeStruct((e, 1), jnp.float32),
        jax.ShapeDtypeStruct((e, 1), jnp.float32),
        jax.ShapeDtypeStruct((1, 128), jnp.float32),
    ]
    grid = (nblocks, nh)
    in_specs = [
        pl.BlockSpec((tb, hc), lambda i, j: (i, j)),
        pl.BlockSpec((e, h), lambda i, j: (0, 0)),
    ]
    out_specs = [
        pl.BlockSpec((tb, e), lambda i, j: (i, 0)),
        pl.BlockSpec((tb, e), lambda i, j: (i, 0)),
        pl.BlockSpec((tb, e), lambda i, j: (i, 0)),
        pl.BlockSpec((_TOPK, tb), lambda i, j: (0, i)),
        pl.BlockSpec((e, 1), lambda i, j: (0, 0)),
        pl.BlockSpec((e, 1), lambda i, j: (0, 0)),
        pl.BlockSpec((1, 128), lambda i, j: (0, 0)),
    ]
    logits, mask, comb, idx, _, _, scal = pl.pallas_call(
        body,
        grid=grid,
        in_specs=in_specs,
        out_specs=out_specs,
        out_shape=out_shape,
        scratch_shapes=[pltpu.VMEM((e, tb), jnp.float32)],
    )(x, w_gate)

    expert_mask = mask.reshape(b, s, e)
    combine_weights = comb.reshape(b, s, e)
    router_logits = logits.reshape(b, s, e)
    expert_indices = idx.T.reshape(b, s, _TOPK)
    aux_loss = scal[0, 2]
    z_loss = scal[0, 3]
    routing_entropy = scal[0, 4]
    return (expert_mask, combine_weights, router_logits, aux_loss, z_loss,
            expert_indices, routing_entropy)


def kernel(hidden_states, W_gate):
    return _router(hidden_states, W_gate)


# speedup vs baseline: 1.2682x; 1.2682x over previous
import functools

import jax
import jax.numpy as jnp
from jax.experimental import pallas as pl
from jax.experimental.pallas import tpu as pltpu

_B, _S, _H = 4, 2048, 4096
_E = 64
_TOPK = 8
_AUX_COEF = 0.01
_Z_COEF = 0.001
_NCOPY = 4


def _start_block_copy(x_hbm, xbuf, sem, block, slot, tb, ncopy):
    rows = tb // ncopy
    for c in range(ncopy):
        pltpu.make_async_copy(
            x_hbm.at[pl.ds(block * tb + c * rows, rows), :],
            xbuf.at[slot, pl.ds(c * rows, rows), :],
            sem.at[slot, c],
        ).start()


def _wait_block_copy(x_hbm, xbuf, sem, block, slot, tb, ncopy):
    rows = tb // ncopy
    for c in range(ncopy):
        pltpu.make_async_copy(
            x_hbm.at[pl.ds(block * tb + c * rows, rows), :],
            xbuf.at[slot, pl.ds(c * rows, rows), :],
            sem.at[slot, c],
        ).wait()


def _router_block_kernel(x_hbm, w_ref, logits_ref, mask_ref, comb_ref,
                         idx_ref, accp_ref, acct_ref, scal_ref,
                         xbuf, sem,
                         *, nblocks, tb, ncopy, topk, aux_coef, z_coef,
                         n_tokens):
    i = pl.program_id(0)
    e = w_ref.shape[0]

    @pl.when(i == 0)
    def _prologue():
        _start_block_copy(x_hbm, xbuf, sem, 0, 0, tb, ncopy)

    @pl.when(i + 1 < nblocks)
    def _prefetch():
        _start_block_copy(x_hbm, xbuf, sem, i + 1, (i + 1) % 2, tb, ncopy)

    _wait_block_copy(x_hbm, xbuf, sem, i, i % 2, tb, ncopy)
    x = xbuf[i % 2]
    w = w_ref[...]
    lt = jax.lax.dot_general(
        w, x, (((1,), (1,)), ((), ())), preferred_element_type=jnp.float32)
    logits_ref[...] = lt.T

    iota_e = jax.lax.broadcasted_iota(jnp.int32, (e, tb), 0)

    m = jnp.max(lt, axis=0, keepdims=True)
    ex = jnp.exp(lt - m)
    se = jnp.sum(ex, axis=0, keepdims=True)
    probs = ex / se
    lse = m + jnp.log(se)
    z_c = jnp.sum(lse * lse)
    ent_c = jnp.sum(lse) - jnp.sum(probs * lt)

    work = lt
    vals = []
    sels = []
    mask_t = jnp.zeros((e, tb), jnp.float32)
    idx_rows = []
    for _ in range(topk):
        mk = jnp.max(work, axis=0, keepdims=True)
        is_max = work == mk
        idx_k = jnp.min(jnp.where(is_max, iota_e, e), axis=0, keepdims=True)
        sel = iota_e == idx_k
        work = jnp.where(sel, -jnp.inf, work)
        vals.append(mk)
        sels.append(sel)
        idx_rows.append(idx_k)
        mask_t = mask_t + sel.astype(jnp.float32)

    idx_ref[...] = jnp.concatenate(idx_rows, axis=0)
    mask_ref[...] = mask_t.T

    topv = jnp.concatenate(vals, axis=0)
    tex = jnp.exp(topv - topv[:1])
    tse = jnp.sum(tex, axis=0, keepdims=True)
    rw = tex / tse
    comb_t = jnp.zeros((e, tb), jnp.float32)
    for k in range(topk):
        comb_t = comb_t + jnp.where(sels[k], rw[k:k + 1, :], 0.0)
    comb_ref[...] = comb_t.T

    @pl.when(i == 0)
    def _init():
        accp_ref[...] = jnp.zeros_like(accp_ref)
        acct_ref[...] = jnp.zeros_like(acct_ref)
        scal_ref[...] = jnp.zeros_like(scal_ref)

    accp_ref[...] += jnp.sum(probs, axis=1, keepdims=True)
    acct_ref[...] += jnp.sum(mask_t, axis=1, keepdims=True)
    lane = jax.lax.broadcasted_iota(jnp.int32, scal_ref.shape, 1)
    scal_ref[...] += (jnp.where(lane == 0, z_c, 0.0)
                      + jnp.where(lane == 1, ent_c, 0.0))

    @pl.when(i == nblocks - 1)
    def _finalize():
        avg_probs = accp_ref[...] / n_tokens
        tpe = acct_ref[...]
        total = jnp.sum(tpe)
        aux = jnp.sum(avg_probs * (tpe / (total + 1e-6))) * e * aux_coef
        srow = scal_ref[...]
        sum_z = jnp.sum(jnp.where(lane == 0, srow, 0.0))
        sum_ent = jnp.sum(jnp.where(lane == 1, srow, 0.0))
        scal_ref[...] = (jnp.where(lane == 2, aux, 0.0)
                         + jnp.where(lane == 3, sum_z / n_tokens * z_coef, 0.0)
                         + jnp.where(lane == 4, sum_ent / n_tokens, 0.0))


@functools.partial(jax.jit, static_argnames=("block_tokens", "ncopy"))
def _router(hidden_states, w_gate, block_tokens=1024, ncopy=_NCOPY):
    b, s, h = hidden_states.shape
    e = w_gate.shape[0]
    t = b * s
    tb = block_tokens
    nblocks = t // tb
    x = hidden_states.reshape(t, h)

    body = functools.partial(
        _router_block_kernel, nblocks=nblocks, tb=tb, ncopy=ncopy,
        topk=_TOPK, aux_coef=_AUX_COEF, z_coef=_Z_COEF, n_tokens=float(t))

    out_shape = [
        jax.ShapeDtypeStruct((t, e), jnp.float32),
        jax.ShapeDtypeStruct((t, e), jnp.float32),
        jax.ShapeDtypeStruct((t, e), jnp.float32),
        jax.ShapeDtypeStruct((_TOPK, t), jnp.int32),
        jax.ShapeDtypeStruct((e, 1), jnp.float32),
        jax.ShapeDtypeStruct((e, 1), jnp.float32),
        jax.ShapeDtypeStruct((1, 128), jnp.float32),
    ]
    grid = (nblocks,)
    in_specs = [
        pl.BlockSpec(memory_space=pl.ANY),
        pl.BlockSpec((e, h), lambda i: (0, 0)),
    ]
    out_specs = [
        pl.BlockSpec((tb, e), lambda i: (i, 0)),
        pl.BlockSpec((tb, e), lambda i: (i, 0)),
        pl.BlockSpec((tb, e), lambda i: (i, 0)),
        pl.BlockSpec((_TOPK, tb), lambda i: (0, i)),
        pl.BlockSpec((e, 1), lambda i: (0, 0)),
        pl.BlockSpec((e, 1), lambda i: (0, 0)),
        pl.BlockSpec((1, 128), lambda i: (0, 0)),
    ]
    logits, mask, comb, idx, _, _, scal = pl.pallas_call(
        body,
        grid=grid,
        in_specs=in_specs,
        out_specs=out_specs,
        out_shape=out_shape,
        scratch_shapes=[
            pltpu.VMEM((2, tb, h), jnp.float32),
            pltpu.SemaphoreType.DMA((2, ncopy)),
        ],
    )(x, w_gate)

    expert_mask = mask.reshape(b, s, e)
    combine_weights = comb.reshape(b, s, e)
    router_logits = logits.reshape(b, s, e)
    expert_indices = idx.T.reshape(b, s, _TOPK)
    aux_loss = scal[0, 2]
    z_loss = scal[0, 3]
    routing_entropy = scal[0, 4]
    return (expert_mask, combine_weights, router_logits, aux_loss, z_loss,
            expert_indices, routing_entropy)


def kernel(hidden_states, W_gate):
    return _router(hidden_states, W_gate)


# speedup vs baseline: 1.2957x; 1.0216x over previous
import functools

import jax
import jax.numpy as jnp
from jax.experimental import pallas as pl

_B, _S, _H = 4, 2048, 4096
_E = 64
_TOPK = 8
_AUX_COEF = 0.01
_Z_COEF = 0.001


def _router_block_kernel(*refs, nblocks, nsplit, topk, aux_coef, z_coef,
                         n_tokens):
    x_refs = refs[:nsplit]
    (w_ref, logits_ref, mask_ref, comb_ref,
     idx_ref, accp_ref, acct_ref, scal_ref) = refs[nsplit:]
    i = pl.program_id(0)
    e = w_ref.shape[0]
    hc = x_refs[0].shape[1]

    lt = None
    for j in range(nsplit):
        part = jax.lax.dot_general(
            w_ref[:, j * hc:(j + 1) * hc], x_refs[j][...],
            (((1,), (1,)), ((), ())), preferred_element_type=jnp.float32)
        lt = part if lt is None else lt + part
    logits_ref[...] = lt.T

    tb = x_refs[0].shape[0]
    iota_e = jax.lax.broadcasted_iota(jnp.int32, (e, tb), 0)

    m = jnp.max(lt, axis=0, keepdims=True)
    ex = jnp.exp(lt - m)
    se = jnp.sum(ex, axis=0, keepdims=True)
    probs = ex / se
    lse = m + jnp.log(se)
    z_c = jnp.sum(lse * lse)
    ent_c = jnp.sum(lse) - jnp.sum(probs * lt)

    work = lt
    vals = []
    sels = []
    mask_t = jnp.zeros((e, tb), jnp.float32)
    idx_rows = []
    for _ in range(topk):
        mk = jnp.max(work, axis=0, keepdims=True)
        is_max = work == mk
        idx_k = jnp.min(jnp.where(is_max, iota_e, e), axis=0, keepdims=True)
        sel = iota_e == idx_k
        work = jnp.where(sel, -jnp.inf, work)
        vals.append(mk)
        sels.append(sel)
        idx_rows.append(idx_k)
        mask_t = mask_t + sel.astype(jnp.float32)

    idx_ref[...] = jnp.concatenate(idx_rows, axis=0)
    mask_ref[...] = mask_t.T

    topv = jnp.concatenate(vals, axis=0)
    tex = jnp.exp(topv - topv[:1])
    tse = jnp.sum(tex, axis=0, keepdims=True)
    rw = tex / tse
    comb_t = jnp.zeros((e, tb), jnp.float32)
    for k in range(topk):
        comb_t = comb_t + jnp.where(sels[k], rw[k:k + 1, :], 0.0)
    comb_ref[...] = comb_t.T

    @pl.when(i == 0)
    def _init():
        accp_ref[...] = jnp.zeros_like(accp_ref)
        acct_ref[...] = jnp.zeros_like(acct_ref)
        scal_ref[...] = jnp.zeros_like(scal_ref)

    accp_ref[...] += jnp.sum(probs, axis=1, keepdims=True)
    acct_ref[...] += jnp.sum(mask_t, axis=1, keepdims=True)
    lane = jax.lax.broadcasted_iota(jnp.int32, scal_ref.shape, 1)
    scal_ref[...] += (jnp.where(lane == 0, z_c, 0.0)
                      + jnp.where(lane == 1, ent_c, 0.0))

    @pl.when(i == nblocks - 1)
    def _finalize():
        avg_probs = accp_ref[...] / n_tokens
        tpe = acct_ref[...]
        total = jnp.sum(tpe)
        aux = jnp.sum(avg_probs * (tpe / (total + 1e-6))) * e * aux_coef
        srow = scal_ref[...]
        sum_z = jnp.sum(jnp.where(lane == 0, srow, 0.0))
        sum_ent = jnp.sum(jnp.where(lane == 1, srow, 0.0))
        scal_ref[...] = (jnp.where(lane == 2, aux, 0.0)
                         + jnp.where(lane == 3, sum_z / n_tokens * z_coef, 0.0)
                         + jnp.where(lane == 4, sum_ent / n_tokens, 0.0))


@functools.partial(jax.jit, static_argnames=("block_tokens", "nsplit"))
def _router(hidden_states, w_gate, block_tokens=1024, nsplit=4):
    b, s, h = hidden_states.shape
    e = w_gate.shape[0]
    t = b * s
    tb = block_tokens
    nblocks = t // tb
    hc = h // nsplit
    x = hidden_states.reshape(t, h)

    body = functools.partial(
        _router_block_kernel, nblocks=nblocks, nsplit=nsplit, topk=_TOPK,
        aux_coef=_AUX_COEF, z_coef=_Z_COEF, n_tokens=float(t))

    out_shape = [
        jax.ShapeDtypeStruct((t, e), jnp.float32),
        jax.ShapeDtypeStruct((t, e), jnp.float32),
        jax.ShapeDtypeStruct((t, e), jnp.float32),
        jax.ShapeDtypeStruct((_TOPK, t), jnp.int32),
        jax.ShapeDtypeStruct((e, 1), jnp.float32),
        jax.ShapeDtypeStruct((e, 1), jnp.float32),
        jax.ShapeDtypeStruct((1, 128), jnp.float32),
    ]
    grid = (nblocks,)
    in_specs = [
        pl.BlockSpec((tb, hc), lambda i, j=j: (i, j)) for j in range(nsplit)
    ] + [
        pl.BlockSpec((e, h), lambda i: (0, 0)),
    ]
    out_specs = [
        pl.BlockSpec((tb, e), lambda i: (i, 0)),
        pl.BlockSpec((tb, e), lambda i: (i, 0)),
        pl.BlockSpec((tb, e), lambda i: (i, 0)),
        pl.BlockSpec((_TOPK, tb), lambda i: (0, i)),
        pl.BlockSpec((e, 1), lambda i: (0, 0)),
        pl.BlockSpec((e, 1), lambda i: (0, 0)),
        pl.BlockSpec((1, 128), lambda i: (0, 0)),
    ]
    logits, mask, comb, idx, _, _, scal = pl.pallas_call(
        body,
        grid=grid,
        in_specs=in_specs,
        out_specs=out_specs,
        out_shape=out_shape,
    )(*([x] * nsplit), w_gate)

    expert_mask = mask.reshape(b, s, e)
    combine_weights = comb.reshape(b, s, e)
    router_logits = logits.reshape(b, s, e)
    expert_indices = idx.T.reshape(b, s, _TOPK)
    aux_loss = scal[0, 2]
    z_loss = scal[0, 3]
    routing_entropy = scal[0, 4]
    return (expert_mask, combine_weights, router_logits, aux_loss, z_loss,
            expert_indices, routing_entropy)


def kernel(hidden_states, W_gate):
    return _router(hidden_states, W_gate)
